# Initial kernel scaffold; baseline (speedup 1.0000x reference)
import jax
import jax.numpy as jnp
from jax.experimental import pallas as pl


def kernel(x_tasks, x_data, ei_tt_to, ei_tt_from, ei_t_read_d, ei_d_read_t, stem_Wt, stem_bt, stem_Wd, stem_bd, ln_t_scale, ln_t_bias, ln_d_scale, ln_d_bias, Wp, bp, Wl, bl, beta_t, beta_d):
    raise NotImplementedError("write your pallas kernel here")


# speedup vs baseline: 4.0699x; 4.0699x over previous
import functools

import jax
import jax.numpy as jnp
from jax import lax
from jax.experimental import pallas as pl
from jax.experimental.pallas import tpu as pltpu
from jax.experimental.pallas import tpu_sc as plsc

N_TASKS = 100000
N_DATA = 100000
N_EDGES = 1600000
HID = 32
N_LAYERS = 2
N_REL = 4

IDX_W = 128
CHUNK_K = 8
N_ITERS = 98
EP = IDX_W * CHUNK_K * N_ITERS * 16
NROWS = EP // IDX_W
ROWS_PER_TILE = NROWS // 16
NACC = 100096
TRASH = 100000
ZROWS = NACC // 16

BLK = 2000
N_BLKS = N_TASKS // BLK

_P_HI = jax.lax.Precision.HIGHEST


def _dot(x, w):
    return jax.lax.dot_general(x, w, (((1,), (0,)), ((), ())),
                               precision=_P_HI,
                               preferred_element_type=jnp.float32)


def _ln_act(x, scale, bias):
    m = jnp.mean(x, axis=-1, keepdims=True)
    v = jnp.mean((x - m) ** 2, axis=-1, keepdims=True)
    return (x - m) / jnp.sqrt(v + 1e-5) * scale + bias


def _silu(x):
    return x * jax.nn.sigmoid(x)


def _stem_body(xt_in, xd_in, wt, bt, wd, bd, lts, ltb, lds, ldb, wp, bp,
               xt_out, xd_out, t_out):
    xt = _silu(_ln_act(_dot(xt_in[...], wt[...]) + bt[...], lts[...], ltb[...]))
    xd = _silu(_ln_act(_dot(xd_in[...], wd[...]) + bd[...], lds[...], ldb[...]))
    xt_out[...] = xt
    xd_out[...] = xd
    for r in range(N_REL):
        src = xt if r < 3 else xd
        p = jnp.maximum(_dot(src, wp[r]) + bp[r], 0.0)
        t_out[r, 0] = p[:, :16]
        t_out[r, 1] = p[:, 16:]


def _stem(x_tasks, x_data, wt, bt, wd, bd, lts, ltb, lds, ldb, wp0, bp0):
    whole = lambda shape: pl.BlockSpec(shape, lambda i: (0,) * len(shape))
    return pl.pallas_call(
        _stem_body,
        grid=(N_BLKS,),
        in_specs=[
            pl.BlockSpec((BLK, 12), lambda i: (i, 0)),
            pl.BlockSpec((BLK, 5), lambda i: (i, 0)),
            whole((12, HID)), whole((1, HID)),
            whole((5, HID)), whole((1, HID)),
            whole((1, HID)), whole((1, HID)),
            whole((1, HID)), whole((1, HID)),
            whole((N_REL, HID, HID)), whole((N_REL, 1, HID)),
        ],
        out_specs=[
            pl.BlockSpec((BLK, HID), lambda i: (i, 0)),
            pl.BlockSpec((BLK, HID), lambda i: (i, 0)),
            pl.BlockSpec((N_REL, 2, BLK, 16), lambda i: (0, 0, i, 0)),
        ],
        out_shape=[
            jax.ShapeDtypeStruct((N_TASKS, HID), jnp.float32),
            jax.ShapeDtypeStruct((N_DATA, HID), jnp.float32),
            jax.ShapeDtypeStruct((N_REL, 2, N_TASKS, 16), jnp.float32),
        ],
        compiler_params=pltpu.CompilerParams(
            dimension_semantics=("arbitrary",)),
    )(x_tasks, x_data, wt, bt, wd, bd, lts, ltb, lds, ldb, wp0, bp0)


def _make_combine_body(last):
    def body(s_in, c_in, xt_in, xd_in, wl, bl, lts, ltb, lds, ldb, bet, bed,
             *rest):
        if last:
            (xt_out, xd_out) = rest
        else:
            (wp, bp, xt_out, xd_out, t_out) = rest
        o = []
        for r in range(N_REL):
            s32 = jnp.concatenate([s_in[r, 0], s_in[r, 1]], axis=-1)
            mean = s32 / jnp.maximum(c_in[r, :, 0:1], 1.0)
            o.append(_dot(mean, wl[r]) + bl[r])
        out_t = (o[0] + o[1] + o[3]) * (1.0 / 3.0)
        out_d = o[2]
        xt = _ln_act(xt_in[...] + bet[0, 0] * _silu(out_t), lts[...], ltb[...])
        xd = _ln_act(xd_in[...] + bed[0, 0] * _silu(out_d), lds[...], ldb[...])
        xt_out[...] = xt
        xd_out[...] = xd
        if not last:
            for r in range(N_REL):
                src = xt if r < 3 else xd
                p = jnp.maximum(_dot(src, wp[r]) + bp[r], 0.0)
                t_out[r, 0] = p[:, :16]
                t_out[r, 1] = p[:, 16:]
    return body


def _combine(last, s, c, xt, xd, wl, bl, lts, ltb, lds, ldb, bet, bed,
             wp=None, bp=None):
    whole = lambda shape: pl.BlockSpec(shape, lambda i: (0,) * len(shape))
    in_specs = [
        pl.BlockSpec((N_REL, 2, BLK, 16), lambda i: (0, 0, i, 0)),
        pl.BlockSpec((N_REL, BLK, 16), lambda i: (0, i, 0)),
        pl.BlockSpec((BLK, HID), lambda i: (i, 0)),
        pl.BlockSpec((BLK, HID), lambda i: (i, 0)),
        whole((N_REL, HID, HID)), whole((N_REL, 1, HID)),
        whole((1, HID)), whole((1, HID)),
        whole((1, HID)), whole((1, HID)),
        whole((1, 1)), whole((1, 1)),
    ]
    args = [s, c, xt, xd, wl, bl, lts, ltb, lds, ldb, bet, bed]
    out_specs = [
        pl.BlockSpec((BLK, HID), lambda i: (i, 0)),
        pl.BlockSpec((BLK, HID), lambda i: (i, 0)),
    ]
    out_shape = [
        jax.ShapeDtypeStruct((N_TASKS, HID), jnp.float32),
        jax.ShapeDtypeStruct((N_DATA, HID), jnp.float32),
    ]
    if not last:
        in_specs += [whole((N_REL, HID, HID)), whole((N_REL, 1, HID))]
        args += [wp, bp]
        out_specs.append(pl.BlockSpec((N_REL, 2, BLK, 16),
                                      lambda i: (0, 0, i, 0)))
        out_shape.append(
            jax.ShapeDtypeStruct((N_REL, 2, N_TASKS, 16), jnp.float32))
    return pl.pallas_call(
        _make_combine_body(last),
        grid=(N_BLKS,),
        in_specs=in_specs,
        out_specs=out_specs,
        out_shape=out_shape,
        compiler_params=pltpu.CompilerParams(
            dimension_semantics=("arbitrary",)),
    )(*args)


def _make_sc_kernel(with_counts):
    mesh = plsc.VectorSubcoreMesh(core_axis_name="c", subcore_axis_name="s",
                                  num_cores=2, num_subcores=16)

    out_type = [jax.ShapeDtypeStruct((N_REL, 2, NACC, 16), jnp.float32)]
    if with_counts:
        out_type.append(jax.ShapeDtypeStruct((N_REL, NACC, 16), jnp.float32))

    scratch = dict(
        src_v=pltpu.VMEM((IDX_W,), jnp.int32),
        dst_v=pltpu.VMEM((IDX_W,), jnp.int32),
        rows_v=pltpu.VMEM((IDX_W, 16), jnp.float32),
        ones_v=pltpu.VMEM((IDX_W, 16), jnp.float32),
        acc=pltpu.VMEM_SHARED((NACC, 16), jnp.float32),
        gsem=pltpu.SemaphoreType.DMA,
        ssem=pltpu.SemaphoreType.DMA,
    )

    @functools.partial(pl.kernel, mesh=mesh, out_type=out_type,
                       scratch_types=scratch,
                       compiler_params=pltpu.CompilerParams(
                           use_tc_tiling_on_sc=False))
    def sc_kernel(t_hbm, src_hbm, dst_hbm, zr_hbm, ones_hbm, *outs,
                  src_v, dst_v, rows_v, ones_v, acc, gsem, ssem):
        if with_counts:
            (s_hbm, c_hbm) = outs
        else:
            (s_hbm,) = outs
        c = lax.axis_index("c")
        s = lax.axis_index("s")

        def zero_acc():
            pltpu.sync_copy(zr_hbm, acc.at[pl.ds(s * ZROWS, ZROWS)])

        for r in range(N_REL):
            zero_acc()
            plsc.subcore_barrier()

            def body(g, carry):
                row0 = s * ROWS_PER_TILE + g
                pltpu.sync_copy(src_hbm.at[r, c, row0], src_v)
                pltpu.sync_copy(dst_hbm.at[r, row0], dst_v)
                pltpu.async_copy(t_hbm.at[src_v], rows_v, gsem).wait()
                pltpu.async_copy(rows_v, acc.at[dst_v], ssem, add=True).wait()
                return carry

            lax.fori_loop(0, ROWS_PER_TILE, body, 0)
            plsc.subcore_barrier()
            pltpu.sync_copy(acc.at[pl.ds(s * ZROWS, ZROWS)],
                            s_hbm.at[r, c, pl.ds(s * ZROWS, ZROWS)])
            plsc.subcore_barrier()

        if with_counts:
            pltpu.sync_copy(ones_hbm, ones_v)
            for k in range(2):
                rel = c * 2 + k
                zero_acc()
                plsc.subcore_barrier()

                def cbody(g, carry):
                    row0 = s * ROWS_PER_TILE + g
                    pltpu.sync_copy(dst_hbm.at[rel, row0], dst_v)
                    pltpu.async_copy(ones_v, acc.at[dst_v], ssem,
                                     add=True).wait()
                    return carry

                lax.fori_loop(0, ROWS_PER_TILE, cbody, 0)
                plsc.subcore_barrier()
                pltpu.sync_copy(acc.at[pl.ds(s * ZROWS, ZROWS)],
                                c_hbm.at[rel, pl.ds(s * ZROWS, ZROWS)])
                plsc.subcore_barrier()

    return sc_kernel


_sc_kernel_cached = functools.lru_cache(maxsize=None)(_make_sc_kernel)


def kernel(x_tasks, x_data, ei_tt_to, ei_tt_from, ei_t_read_d, ei_d_read_t,
           stem_Wt, stem_bt, stem_Wd, stem_bd,
           ln_t_scale, ln_t_bias, ln_d_scale, ln_d_bias,
           Wp, bp, Wl, bl, beta_t, beta_d):
    f32 = jnp.float32
    pad = jnp.stack([jnp.zeros((EP - N_EDGES,), jnp.int32),
                     jnp.full((EP - N_EDGES,), TRASH, jnp.int32)])
    srcs, dsts = [], []
    for r, e in enumerate((ei_tt_to, ei_tt_from, ei_t_read_d, ei_d_read_t)):
        e = jnp.concatenate([e.astype(jnp.int32), pad], axis=1)
        src, dst = e[0], e[1]
        base = r * 2 * N_TASKS
        srcs.append(jnp.stack([src + base, src + base + N_TASKS])
                    .reshape(2, NROWS, IDX_W))
        dsts.append(dst.reshape(NROWS, IDX_W))
    src_idx = jnp.stack(srcs)
    dst_idx = jnp.stack(dsts)

    zr = jnp.zeros((ZROWS, 16), f32)
    ones = jnp.ones((IDX_W, 16), f32)

    r2 = lambda a: a.reshape(1, HID)
    bp_r = bp.reshape(N_LAYERS, N_REL, 1, HID)
    bl_r = bl.reshape(N_LAYERS, N_REL, 1, HID)

    xt, xd, t0 = _stem(
        x_tasks, x_data, stem_Wt, r2(stem_bt), stem_Wd, r2(stem_bd),
        r2(ln_t_scale[0]), r2(ln_t_bias[0]),
        r2(ln_d_scale[0]), r2(ln_d_bias[0]),
        Wp[0], bp_r[0])

    s0, cdeg = _sc_kernel_cached(True)(
        t0.reshape(N_REL * 2 * N_TASKS, 16), src_idx, dst_idx, zr, ones)

    xt, xd, t1 = _combine(
        False, s0, cdeg, xt, xd, Wl[0], bl_r[0],
        r2(ln_t_scale[1]), r2(ln_t_bias[1]),
        r2(ln_d_scale[1]), r2(ln_d_bias[1]),
        beta_t[0].reshape(1, 1), beta_d[0].reshape(1, 1),
        Wp[1], bp_r[1])

    (s1,) = _sc_kernel_cached(False)(
        t1.reshape(N_REL * 2 * N_TASKS, 16), src_idx, dst_idx, zr, ones)

    xt, xd = _combine(
        True, s1, cdeg, xt, xd, Wl[1], bl_r[1],
        r2(ln_t_scale[2]), r2(ln_t_bias[2]),
        r2(ln_d_scale[2]), r2(ln_d_bias[2]),
        beta_t[1].reshape(1, 1), beta_d[1].reshape(1, 1))

    return jnp.concatenate([xt, xd], axis=0)


# speedup vs baseline: 11.8817x; 2.9194x over previous
import functools

import jax
import jax.numpy as jnp
from jax import lax
from jax.experimental import pallas as pl
from jax.experimental.pallas import tpu as pltpu
from jax.experimental.pallas import tpu_sc as plsc

N_TASKS = 100000
N_DATA = 100000
N_EDGES = 1600000
HID = 32
N_LAYERS = 2
N_REL = 4

IDX_W = 128
CHUNK_K = 4
ROWS_PER_TILE = 784
EP = IDX_W * ROWS_PER_TILE * 16
NROWS = EP // IDX_W
N_ITERS = ROWS_PER_TILE // CHUNK_K
NACC = 100096
TRASH = 100000
ZROWS = NACC // 16

BLK = 2000
N_BLKS = N_TASKS // BLK

_P_HI = jax.lax.Precision.HIGHEST


def _dot(x, w):
    return jax.lax.dot_general(x, w, (((1,), (0,)), ((), ())),
                               precision=_P_HI,
                               preferred_element_type=jnp.float32)


def _ln_act(x, scale, bias):
    m = jnp.mean(x, axis=-1, keepdims=True)
    v = jnp.mean((x - m) ** 2, axis=-1, keepdims=True)
    return (x - m) / jnp.sqrt(v + 1e-5) * scale + bias


def _silu(x):
    return x * jax.nn.sigmoid(x)


def _stem_body(xt_in, xd_in, wt, bt, wd, bd, lts, ltb, lds, ldb, wp, bp,
               xt_out, xd_out, t_out):
    xt = _silu(_ln_act(_dot(xt_in[...], wt[...]) + bt[...], lts[...], ltb[...]))
    xd = _silu(_ln_act(_dot(xd_in[...], wd[...]) + bd[...], lds[...], ldb[...]))
    xt_out[...] = xt
    xd_out[...] = xd
    for r in range(N_REL):
        src = xt if r < 3 else xd
        p = jnp.maximum(_dot(src, wp[r]) + bp[r], 0.0)
        t_out[r, 0] = p[:, :16]
        t_out[r, 1] = p[:, 16:]


def _stem(x_tasks, x_data, wt, bt, wd, bd, lts, ltb, lds, ldb, wp0, bp0):
    whole = lambda shape: pl.BlockSpec(shape, lambda i: (0,) * len(shape))
    return pl.pallas_call(
        _stem_body,
        grid=(N_BLKS,),
        in_specs=[
            pl.BlockSpec((BLK, 12), lambda i: (i, 0)),
            pl.BlockSpec((BLK, 5), lambda i: (i, 0)),
            whole((12, HID)), whole((1, HID)),
            whole((5, HID)), whole((1, HID)),
            whole((1, HID)), whole((1, HID)),
            whole((1, HID)), whole((1, HID)),
            whole((N_REL, HID, HID)), whole((N_REL, 1, HID)),
        ],
        out_specs=[
            pl.BlockSpec((BLK, HID), lambda i: (i, 0)),
            pl.BlockSpec((BLK, HID), lambda i: (i, 0)),
            pl.BlockSpec((N_REL, 2, BLK, 16), lambda i: (0, 0, i, 0)),
        ],
        out_shape=[
            jax.ShapeDtypeStruct((N_TASKS, HID), jnp.float32),
            jax.ShapeDtypeStruct((N_DATA, HID), jnp.float32),
            jax.ShapeDtypeStruct((N_REL, 2, N_TASKS, 16), jnp.float32),
        ],
        compiler_params=pltpu.CompilerParams(
            dimension_semantics=("arbitrary",)),
    )(x_tasks, x_data, wt, bt, wd, bd, lts, ltb, lds, ldb, wp0, bp0)


def _make_combine_body(last):
    def body(s_in, c_in, xt_in, xd_in, wl, bl, lts, ltb, lds, ldb, bet, bed,
             *rest):
        if last:
            (xt_out, xd_out) = rest
        else:
            (wp, bp, xt_out, xd_out, t_out) = rest
        o = []
        for r in range(N_REL):
            s32 = jnp.concatenate([s_in[r, 0], s_in[r, 1]], axis=-1)
            mean = s32 / jnp.maximum(c_in[r, :, 0:1], 1.0)
            o.append(_dot(mean, wl[r]) + bl[r])
        out_t = (o[0] + o[1] + o[3]) * (1.0 / 3.0)
        out_d = o[2]
        xt = _ln_act(xt_in[...] + bet[0, 0] * _silu(out_t), lts[...], ltb[...])
        xd = _ln_act(xd_in[...] + bed[0, 0] * _silu(out_d), lds[...], ldb[...])
        xt_out[...] = xt
        xd_out[...] = xd
        if not last:
            for r in range(N_REL):
                src = xt if r < 3 else xd
                p = jnp.maximum(_dot(src, wp[r]) + bp[r], 0.0)
                t_out[r, 0] = p[:, :16]
                t_out[r, 1] = p[:, 16:]
    return body


def _combine(last, s, c, xt, xd, wl, bl, lts, ltb, lds, ldb, bet, bed,
             wp=None, bp=None):
    whole = lambda shape: pl.BlockSpec(shape, lambda i: (0,) * len(shape))
    in_specs = [
        pl.BlockSpec((N_REL, 2, BLK, 16), lambda i: (0, 0, i, 0)),
        pl.BlockSpec((N_REL, BLK, 16), lambda i: (0, i, 0)),
        pl.BlockSpec((BLK, HID), lambda i: (i, 0)),
        pl.BlockSpec((BLK, HID), lambda i: (i, 0)),
        whole((N_REL, HID, HID)), whole((N_REL, 1, HID)),
        whole((1, HID)), whole((1, HID)),
        whole((1, HID)), whole((1, HID)),
        whole((1, 1)), whole((1, 1)),
    ]
    args = [s, c, xt, xd, wl, bl, lts, ltb, lds, ldb, bet, bed]
    out_specs = [
        pl.BlockSpec((BLK, HID), lambda i: (i, 0)),
        pl.BlockSpec((BLK, HID), lambda i: (i, 0)),
    ]
    out_shape = [
        jax.ShapeDtypeStruct((N_TASKS, HID), jnp.float32),
        jax.ShapeDtypeStruct((N_DATA, HID), jnp.float32),
    ]
    if not last:
        in_specs += [whole((N_REL, HID, HID)), whole((N_REL, 1, HID))]
        args += [wp, bp]
        out_specs.append(pl.BlockSpec((N_REL, 2, BLK, 16),
                                      lambda i: (0, 0, i, 0)))
        out_shape.append(
            jax.ShapeDtypeStruct((N_REL, 2, N_TASKS, 16), jnp.float32))
    return pl.pallas_call(
        _make_combine_body(last),
        grid=(N_BLKS,),
        in_specs=in_specs,
        out_specs=out_specs,
        out_shape=out_shape,
        compiler_params=pltpu.CompilerParams(
            dimension_semantics=("arbitrary",)),
    )(*args)


def _make_sc_kernel(with_counts):
    mesh = plsc.VectorSubcoreMesh(core_axis_name="c", subcore_axis_name="s",
                                  num_cores=2, num_subcores=16)

    out_type = [jax.ShapeDtypeStruct((N_REL, 2, NACC, 16), jnp.float32)]
    if with_counts:
        out_type.append(jax.ShapeDtypeStruct((N_REL, NACC, 16), jnp.float32))

    scratch = dict(
        src_v=pltpu.VMEM((2, CHUNK_K, IDX_W), jnp.int32),
        dst_v=pltpu.VMEM((2, CHUNK_K, IDX_W), jnp.int32),
        rows_v=pltpu.VMEM((2, CHUNK_K, IDX_W, 16), jnp.float32),
        ones_v=pltpu.VMEM((IDX_W, 16), jnp.float32),
        acc=pltpu.VMEM_SHARED((NACC, 16), jnp.float32),
        isem=pltpu.SemaphoreType.DMA,
        gsem=pltpu.SemaphoreType.DMA,
        ssem=pltpu.SemaphoreType.DMA,
    )

    @functools.partial(pl.kernel, mesh=mesh, out_type=out_type,
                       scratch_types=scratch,
                       compiler_params=pltpu.CompilerParams(
                           use_tc_tiling_on_sc=False))
    def sc_kernel(t_hbm, src_hbm, dst_hbm, zr_hbm, ones_hbm, *outs,
                  src_v, dst_v, rows_v, ones_v, acc, isem, gsem, ssem):
        if with_counts:
            (s_hbm, c_hbm) = outs
        else:
            (s_hbm,) = outs
        c = lax.axis_index("c")
        s = lax.axis_index("s")
        base = s * ROWS_PER_TILE
        n_half = N_ITERS // 2

        def zero_acc():
            pltpu.sync_copy(zr_hbm, acc.at[pl.ds(s * ZROWS, ZROWS)])

        for r in range(N_REL):
            def idx_copies(g, p):
                row0 = base + g * CHUNK_K
                return (
                    pltpu.make_async_copy(
                        src_hbm.at[r, c, pl.ds(row0, CHUNK_K)],
                        src_v.at[p], isem),
                    pltpu.make_async_copy(
                        dst_hbm.at[r, pl.ds(row0, CHUNK_K)],
                        dst_v.at[p], isem),
                )

            def issue_idx(g, p):
                for d in idx_copies(g, p):
                    d.start()

            def wait_idx(g, p):
                for d in idx_copies(g, p):
                    d.wait()

            def fire_gathers(p):
                return [pltpu.async_copy(t_hbm.at[src_v.at[p, j]],
                                         rows_v.at[p, j], gsem)
                        for j in range(CHUNK_K)]

            def fire_scatters(p):
                for j in range(CHUNK_K):
                    pltpu.async_copy(rows_v.at[p, j], acc.at[dst_v.at[p, j]],
                                     ssem, add=True)

            def drain_scatters(p):
                for j in range(CHUNK_K):
                    pltpu.make_async_copy(rows_v.at[p, j],
                                          acc.at[dst_v.at[p, j]],
                                          ssem).wait()

            zero_acc()
            plsc.subcore_barrier()
            issue_idx(0, 0)

            def body2(h, carry):
                wait_idx(2 * h, 0)
                gh = fire_gathers(0)

                @pl.when(h >= 1)
                def _():
                    drain_scatters(1)
                issue_idx(2 * h + 1, 1)
                for d in gh:
                    d.wait()
                fire_scatters(0)
                wait_idx(2 * h + 1, 1)
                gh1 = fire_gathers(1)
                drain_scatters(0)

                @pl.when(h <= n_half - 2)
                def _():
                    issue_idx(2 * h + 2, 0)
                for d in gh1:
                    d.wait()
                fire_scatters(1)
                return carry

            lax.fori_loop(0, n_half, body2, 0)
            drain_scatters(1)
            plsc.subcore_barrier()
            pltpu.sync_copy(acc.at[pl.ds(s * ZROWS, ZROWS)],
                            s_hbm.at[r, c, pl.ds(s * ZROWS, ZROWS)])
            plsc.subcore_barrier()

        if with_counts:
            pltpu.sync_copy(ones_hbm, ones_v)
            for k in range(2):
                rel = c * 2 + k

                def cidx_copy(g, p):
                    row0 = base + g * CHUNK_K
                    return pltpu.make_async_copy(
                        dst_hbm.at[rel, pl.ds(row0, CHUNK_K)],
                        dst_v.at[p], isem)

                def fire_cscat(p):
                    for j in range(CHUNK_K):
                        pltpu.async_copy(ones_v, acc.at[dst_v.at[p, j]],
                                         ssem, add=True)

                def drain_cscat(p):
                    for j in range(CHUNK_K):
                        pltpu.make_async_copy(ones_v,
                                              acc.at[dst_v.at[p, j]],
                                              ssem).wait()

                zero_acc()
                plsc.subcore_barrier()
                cidx_copy(0, 0).start()

                def cbody2(h, carry):
                    cidx_copy(2 * h, 0).wait()

                    @pl.when(h >= 1)
                    def _():
                        drain_cscat(1)
                    cidx_copy(2 * h + 1, 1).start()
                    fire_cscat(0)
                    cidx_copy(2 * h + 1, 1).wait()
                    drain_cscat(0)

                    @pl.when(h <= n_half - 2)
                    def _():
                        cidx_copy(2 * h + 2, 0).start()
                    fire_cscat(1)
                    return carry

                lax.fori_loop(0, n_half, cbody2, 0)
                drain_cscat(1)
                plsc.subcore_barrier()
                pltpu.sync_copy(acc.at[pl.ds(s * ZROWS, ZROWS)],
                                c_hbm.at[rel, pl.ds(s * ZROWS, ZROWS)])
                plsc.subcore_barrier()

    return sc_kernel


_sc_kernel_cached = functools.lru_cache(maxsize=None)(_make_sc_kernel)


def kernel(x_tasks, x_data, ei_tt_to, ei_tt_from, ei_t_read_d, ei_d_read_t,
           stem_Wt, stem_bt, stem_Wd, stem_bd,
           ln_t_scale, ln_t_bias, ln_d_scale, ln_d_bias,
           Wp, bp, Wl, bl, beta_t, beta_d):
    f32 = jnp.float32
    pad = jnp.stack([jnp.zeros((EP - N_EDGES,), jnp.int32),
                     jnp.full((EP - N_EDGES,), TRASH, jnp.int32)])
    srcs, dsts = [], []
    for r, e in enumerate((ei_tt_to, ei_tt_from, ei_t_read_d, ei_d_read_t)):
        e = jnp.concatenate([e.astype(jnp.int32), pad], axis=1)
        src, dst = e[0], e[1]
        base = r * 2 * N_TASKS
        srcs.append(jnp.stack([src + base, src + base + N_TASKS])
                    .reshape(2, NROWS, IDX_W))
        dsts.append(dst.reshape(NROWS, IDX_W))
    src_idx = jnp.stack(srcs)
    dst_idx = jnp.stack(dsts)

    zr = jnp.zeros((ZROWS, 16), f32)
    ones = jnp.ones((IDX_W, 16), f32)

    r2 = lambda a: a.reshape(1, HID)
    bp_r = bp.reshape(N_LAYERS, N_REL, 1, HID)
    bl_r = bl.reshape(N_LAYERS, N_REL, 1, HID)

    xt, xd, t0 = _stem(
        x_tasks, x_data, stem_Wt, r2(stem_bt), stem_Wd, r2(stem_bd),
        r2(ln_t_scale[0]), r2(ln_t_bias[0]),
        r2(ln_d_scale[0]), r2(ln_d_bias[0]),
        Wp[0], bp_r[0])

    s0, cdeg = _sc_kernel_cached(True)(
        t0.reshape(N_REL * 2 * N_TASKS, 16), src_idx, dst_idx, zr, ones)

    xt, xd, t1 = _combine(
        False, s0, cdeg, xt, xd, Wl[0], bl_r[0],
        r2(ln_t_scale[1]), r2(ln_t_bias[1]),
        r2(ln_d_scale[1]), r2(ln_d_bias[1]),
        beta_t[0].reshape(1, 1), beta_d[0].reshape(1, 1),
        Wp[1], bp_r[1])

    (s1,) = _sc_kernel_cached(False)(
        t1.reshape(N_REL * 2 * N_TASKS, 16), src_idx, dst_idx, zr, ones)

    xt, xd = _combine(
        True, s1, cdeg, xt, xd, Wl[1], bl_r[1],
        r2(ln_t_scale[2]), r2(ln_t_bias[2]),
        r2(ln_d_scale[2]), r2(ln_d_bias[2]),
        beta_t[1].reshape(1, 1), beta_d[1].reshape(1, 1))

    return jnp.concatenate([xt, xd], axis=0)


# speedup vs baseline: 12.0663x; 1.0155x over previous
import functools

import jax
import jax.numpy as jnp
from jax import lax
from jax.experimental import pallas as pl
from jax.experimental.pallas import tpu as pltpu
from jax.experimental.pallas import tpu_sc as plsc

N_TASKS = 100000
N_DATA = 100000
N_EDGES = 1600000
HID = 32
N_LAYERS = 2
N_REL = 4

IDX_W = 128
CHUNK_K = 4
ROWS_PER_TILE = 784
EP = IDX_W * ROWS_PER_TILE * 16
NROWS = EP // IDX_W
N_ITERS = ROWS_PER_TILE // CHUNK_K
NACC = 104000
TRASH = 100000
ZSLICE = NACC // 4

BLK = 2000
N_BLKS = N_TASKS // BLK

_P_HI = jax.lax.Precision.HIGHEST


def _dot(x, w):
    return jax.lax.dot_general(x, w, (((1,), (0,)), ((), ())),
                               precision=_P_HI,
                               preferred_element_type=jnp.float32)


def _ln_act(x, scale, bias):
    m = jnp.mean(x, axis=-1, keepdims=True)
    v = jnp.mean((x - m) ** 2, axis=-1, keepdims=True)
    return (x - m) / jnp.sqrt(v + 1e-5) * scale + bias


def _silu(x):
    return x * jax.nn.sigmoid(x)


def _stem_body(xt_in, xd_in, wt, bt, wd, bd, lts, ltb, lds, ldb, wpt, bpt,
               wpd, bpd, xt_out, xd_out, t_out):
    xt = _silu(_ln_act(_dot(xt_in[...], wt[...]) + bt[...], lts[...], ltb[...]))
    xd = _silu(_ln_act(_dot(xd_in[...], wd[...]) + bd[...], lds[...], ldb[...]))
    xt_out[...] = xt
    xd_out[...] = xd
    pt = jnp.maximum(_dot(xt, wpt[...]) + bpt[...], 0.0)
    pd = jnp.maximum(_dot(xd, wpd[...]) + bpd[...], 0.0)
    t_out[...] = jnp.concatenate([pt, pd], axis=-1)


def _stem(x_tasks, x_data, wt, bt, wd, bd, lts, ltb, lds, ldb, wpt, bpt,
          wpd, bpd):
    whole = lambda shape: pl.BlockSpec(shape, lambda i: (0,) * len(shape))
    return pl.pallas_call(
        _stem_body,
        grid=(N_BLKS,),
        in_specs=[
            pl.BlockSpec((BLK, 12), lambda i: (i, 0)),
            pl.BlockSpec((BLK, 5), lambda i: (i, 0)),
            whole((12, HID)), whole((1, HID)),
            whole((5, HID)), whole((1, HID)),
            whole((1, HID)), whole((1, HID)),
            whole((1, HID)), whole((1, HID)),
            whole((HID, 96)), whole((1, 96)),
            whole((HID, HID)), whole((1, HID)),
        ],
        out_specs=[
            pl.BlockSpec((BLK, HID), lambda i: (i, 0)),
            pl.BlockSpec((BLK, HID), lambda i: (i, 0)),
            pl.BlockSpec((BLK, 128), lambda i: (i, 0)),
        ],
        out_shape=[
            jax.ShapeDtypeStruct((N_TASKS, HID), jnp.float32),
            jax.ShapeDtypeStruct((N_DATA, HID), jnp.float32),
            jax.ShapeDtypeStruct((N_TASKS, 128), jnp.float32),
        ],
        compiler_params=pltpu.CompilerParams(
            dimension_semantics=("arbitrary",)),
    )(x_tasks, x_data, wt, bt, wd, bd, lts, ltb, lds, ldb, wpt, bpt,
      wpd, bpd)


def _make_combine_body(last):
    def body(s_in, c_in, xt_in, xd_in, wlt, blt, wld, bld,
             lts, ltb, lds, ldb, bet, bed, *rest):
        if last:
            (xt_out, xd_out) = rest
        else:
            (wpt, bpt, wpd, bpd, xt_out, xd_out, t_out) = rest
        mean = s_in[...] / jnp.maximum(c_in[...], 1.0)
        out_t = _dot(mean, wlt[...]) + blt[...]
        out_d = _dot(mean, wld[...]) + bld[...]
        xt = _ln_act(xt_in[...] + bet[0, 0] * _silu(out_t), lts[...], ltb[...])
        xd = _ln_act(xd_in[...] + bed[0, 0] * _silu(out_d), lds[...], ldb[...])
        xt_out[...] = xt
        xd_out[...] = xd
        if not last:
            pt = jnp.maximum(_dot(xt, wpt[...]) + bpt[...], 0.0)
            pd = jnp.maximum(_dot(xd, wpd[...]) + bpd[...], 0.0)
            t_out[...] = jnp.concatenate([pt, pd], axis=-1)
    return body


def _combine(last, s, c, xt, xd, wlt, blt, wld, bld, lts, ltb, lds, ldb,
             bet, bed, wpt=None, bpt=None, wpd=None, bpd=None):
    whole = lambda shape: pl.BlockSpec(shape, lambda i: (0,) * len(shape))
    in_specs = [
        pl.BlockSpec((BLK, 128), lambda i: (i, 0)),
        pl.BlockSpec((BLK, 128), lambda i: (i, 0)),
        pl.BlockSpec((BLK, HID), lambda i: (i, 0)),
        pl.BlockSpec((BLK, HID), lambda i: (i, 0)),
        whole((128, HID)), whole((1, HID)),
        whole((128, HID)), whole((1, HID)),
        whole((1, HID)), whole((1, HID)),
        whole((1, HID)), whole((1, HID)),
        whole((1, 1)), whole((1, 1)),
    ]
    args = [s, c, xt, xd, wlt, blt, wld, bld, lts, ltb, lds, ldb, bet, bed]
    out_specs = [
        pl.BlockSpec((BLK, HID), lambda i: (i, 0)),
        pl.BlockSpec((BLK, HID), lambda i: (i, 0)),
    ]
    out_shape = [
        jax.ShapeDtypeStruct((N_TASKS, HID), jnp.float32),
        jax.ShapeDtypeStruct((N_DATA, HID), jnp.float32),
    ]
    if not last:
        in_specs += [whole((HID, 96)), whole((1, 96)),
                     whole((HID, HID)), whole((1, HID))]
        args += [wpt, bpt, wpd, bpd]
        out_specs.append(pl.BlockSpec((BLK, 128), lambda i: (i, 0)))
        out_shape.append(
            jax.ShapeDtypeStruct((N_TASKS, 128), jnp.float32))
    return pl.pallas_call(
        _make_combine_body(last),
        grid=(N_BLKS,),
        in_specs=in_specs,
        out_specs=out_specs,
        out_shape=out_shape,
        compiler_params=pltpu.CompilerParams(
            dimension_semantics=("arbitrary",)),
    )(*args)


def _make_sc_kernel(with_counts):
    mesh = plsc.VectorSubcoreMesh(core_axis_name="c", subcore_axis_name="s",
                                  num_cores=2, num_subcores=16)

    out_type = [jax.ShapeDtypeStruct((NACC, 8, 16), jnp.float32)]
    if with_counts:
        out_type.append(jax.ShapeDtypeStruct((NACC, 8, 16), jnp.float32))

    scratch = dict(
        src_v=pltpu.VMEM((2, CHUNK_K, IDX_W), jnp.int32),
        dst_v=pltpu.VMEM((2, CHUNK_K, IDX_W), jnp.int32),
        rows_v=pltpu.VMEM((2, CHUNK_K, IDX_W, 16), jnp.float32),
        ones_v=pltpu.VMEM((IDX_W, 16), jnp.float32),
        acc=pltpu.VMEM_SHARED((NACC, 16), jnp.float32),
        isem=pltpu.SemaphoreType.DMA,
        gsem=pltpu.SemaphoreType.DMA,
        ssem=pltpu.SemaphoreType.DMA,
    )

    @functools.partial(pl.kernel, mesh=mesh, out_type=out_type,
                       scratch_types=scratch,
                       compiler_params=pltpu.CompilerParams(
                           use_tc_tiling_on_sc=False))
    def sc_kernel(t_hbm, src_hbm, dst_hbm, zr_hbm, ones_hbm, *outs,
                  src_v, dst_v, rows_v, ones_v, acc, isem, gsem, ssem):
        if with_counts:
            (s_hbm, c_hbm) = outs
        else:
            (s_hbm,) = outs
        c = lax.axis_index("c")
        s = lax.axis_index("s")
        base = s * ROWS_PER_TILE
        n_half = N_ITERS // 2

        def zero_acc():
            @pl.when(s < 4)
            def _():
                pltpu.sync_copy(zr_hbm, acc.at[pl.ds(s * ZSLICE, ZSLICE)])

        for r in range(N_REL):
            def idx_copies(g, p):
                row0 = base + g * CHUNK_K
                return (
                    pltpu.make_async_copy(
                        src_hbm.at[r, c, pl.ds(row0, CHUNK_K)],
                        src_v.at[p], isem),
                    pltpu.make_async_copy(
                        dst_hbm.at[r, pl.ds(row0, CHUNK_K)],
                        dst_v.at[p], isem),
                )

            def issue_idx(g, p):
                for d in idx_copies(g, p):
                    d.start()

            def wait_idx(g, p):
                for d in idx_copies(g, p):
                    d.wait()

            def fire_gathers(p):
                return [pltpu.async_copy(t_hbm.at[src_v.at[p, j]],
                                         rows_v.at[p, j], gsem)
                        for j in range(CHUNK_K)]

            def fire_scatters(p):
                for j in range(CHUNK_K):
                    pltpu.async_copy(rows_v.at[p, j], acc.at[dst_v.at[p, j]],
                                     ssem, add=True)

            def drain_scatters(p):
                for j in range(CHUNK_K):
                    pltpu.make_async_copy(rows_v.at[p, j],
                                          acc.at[dst_v.at[p, j]],
                                          ssem).wait()

            zero_acc()
            plsc.subcore_barrier()
            issue_idx(0, 0)

            def body2(h, carry):
                wait_idx(2 * h, 0)
                gh = fire_gathers(0)

                @pl.when(h >= 1)
                def _():
                    drain_scatters(1)
                issue_idx(2 * h + 1, 1)
                for d in gh:
                    d.wait()
                fire_scatters(0)
                wait_idx(2 * h + 1, 1)
                gh1 = fire_gathers(1)
                drain_scatters(0)

                @pl.when(h <= n_half - 2)
                def _():
                    issue_idx(2 * h + 2, 0)
                for d in gh1:
                    d.wait()
                fire_scatters(1)
                return carry

            lax.fori_loop(0, n_half, body2, 0)
            drain_scatters(1)
            plsc.subcore_barrier()
            @pl.when(s < 4)
            def _():
                pltpu.sync_copy(acc.at[pl.ds(s * ZSLICE, ZSLICE)],
                                s_hbm.at[pl.ds(s * ZSLICE, ZSLICE),
                                         2 * r + c])
            plsc.subcore_barrier()

        if with_counts:
            pltpu.sync_copy(ones_hbm, ones_v)
            for k in range(2):
                rel = c * 2 + k

                def cidx_copy(g, p):
                    row0 = base + g * CHUNK_K
                    return pltpu.make_async_copy(
                        dst_hbm.at[rel, pl.ds(row0, CHUNK_K)],
                        dst_v.at[p], isem)

                def fire_cscat(p):
                    for j in range(CHUNK_K):
                        pltpu.async_copy(ones_v, acc.at[dst_v.at[p, j]],
                                         ssem, add=True)

                def drain_cscat(p):
                    for j in range(CHUNK_K):
                        pltpu.make_async_copy(ones_v,
                                              acc.at[dst_v.at[p, j]],
                                              ssem).wait()

                zero_acc()
                plsc.subcore_barrier()
                cidx_copy(0, 0).start()

                def cbody2(h, carry):
                    cidx_copy(2 * h, 0).wait()

                    @pl.when(h >= 1)
                    def _():
                        drain_cscat(1)
                    cidx_copy(2 * h + 1, 1).start()
                    fire_cscat(0)
                    cidx_copy(2 * h + 1, 1).wait()
                    drain_cscat(0)

                    @pl.when(h <= n_half - 2)
                    def _():
                        cidx_copy(2 * h + 2, 0).start()
                    fire_cscat(1)
                    return carry

                lax.fori_loop(0, n_half, cbody2, 0)
                drain_cscat(1)
                plsc.subcore_barrier()

                @pl.when(s < 4)
                def _():
                    pltpu.sync_copy(acc.at[pl.ds(s * ZSLICE, ZSLICE)],
                                    c_hbm.at[pl.ds(s * ZSLICE, ZSLICE),
                                             2 * rel])
                    pltpu.sync_copy(acc.at[pl.ds(s * ZSLICE, ZSLICE)],
                                    c_hbm.at[pl.ds(s * ZSLICE, ZSLICE),
                                             2 * rel + 1])
                plsc.subcore_barrier()

    return sc_kernel


_sc_kernel_cached = functools.lru_cache(maxsize=None)(_make_sc_kernel)


def kernel(x_tasks, x_data, ei_tt_to, ei_tt_from, ei_t_read_d, ei_d_read_t,
           stem_Wt, stem_bt, stem_Wd, stem_bd,
           ln_t_scale, ln_t_bias, ln_d_scale, ln_d_bias,
           Wp, bp, Wl, bl, beta_t, beta_d):
    f32 = jnp.float32
    pad = jnp.stack([jnp.zeros((EP - N_EDGES,), jnp.int32),
                     jnp.full((EP - N_EDGES,), TRASH, jnp.int32)])
    srcs, dsts = [], []
    for r, e in enumerate((ei_tt_to, ei_tt_from, ei_t_read_d, ei_d_read_t)):
        e = jnp.concatenate([e.astype(jnp.int32), pad], axis=1)
        src, dst = e[0], e[1]
        q = src * 8 + 2 * r
        srcs.append(jnp.stack([q, q + 1]).reshape(2, NROWS, IDX_W))
        dsts.append(dst.reshape(NROWS, IDX_W))
    src_idx = jnp.stack(srcs)
    dst_idx = jnp.stack(dsts)

    zr = jnp.zeros((ZSLICE, 16), f32)
    ones = jnp.ones((IDX_W, 16), f32)

    r2 = lambda a: a.reshape(1, HID)
    z32 = jnp.zeros((HID, HID), f32)
    wpt, bpt, wpd, bpd, wlt, blt, wld, bld = [], [], [], [], [], [], [], []
    for l in range(N_LAYERS):
        wpt.append(jnp.concatenate([Wp[l, 0], Wp[l, 1], Wp[l, 2]], axis=1))
        bpt.append(jnp.concatenate([bp[l, 0], bp[l, 1], bp[l, 2]])
                   .reshape(1, 96))
        wpd.append(Wp[l, 3])
        bpd.append(bp[l, 3].reshape(1, HID))
        wlt.append(jnp.concatenate(
            [Wl[l, 0], Wl[l, 1], z32, Wl[l, 3]], axis=0) / 3.0)
        blt.append(((bl[l, 0] + bl[l, 1] + bl[l, 3]) / 3.0).reshape(1, HID))
        wld.append(jnp.concatenate([z32, z32, Wl[l, 2], z32], axis=0))
        bld.append(bl[l, 2].reshape(1, HID))

    xt, xd, t0 = _stem(
        x_tasks, x_data, stem_Wt, r2(stem_bt), stem_Wd, r2(stem_bd),
        r2(ln_t_scale[0]), r2(ln_t_bias[0]),
        r2(ln_d_scale[0]), r2(ln_d_bias[0]),
        wpt[0], bpt[0], wpd[0], bpd[0])

    s0, cdeg = _sc_kernel_cached(True)(
        t0.reshape(8 * N_TASKS, 16), src_idx, dst_idx, zr, ones)
    s0 = s0.reshape(NACC, 128)
    cdeg = cdeg.reshape(NACC, 128)

    xt, xd, t1 = _combine(
        False, s0, cdeg, xt, xd, wlt[0], blt[0], wld[0], bld[0],
        r2(ln_t_scale[1]), r2(ln_t_bias[1]),
        r2(ln_d_scale[1]), r2(ln_d_bias[1]),
        beta_t[0].reshape(1, 1), beta_d[0].reshape(1, 1),
        wpt[1], bpt[1], wpd[1], bpd[1])

    (s1,) = _sc_kernel_cached(False)(
        t1.reshape(8 * N_TASKS, 16), src_idx, dst_idx, zr, ones)
    s1 = s1.reshape(NACC, 128)

    xt, xd = _combine(
        True, s1, cdeg, xt, xd, wlt[1], blt[1], wld[1], bld[1],
        r2(ln_t_scale[2]), r2(ln_t_bias[2]),
        r2(ln_d_scale[2]), r2(ln_d_bias[2]),
        beta_t[1].reshape(1, 1), beta_d[1].reshape(1, 1))

    return jnp.concatenate([xt, xd], axis=0)


# speedup vs baseline: 13.0749x; 1.0836x over previous
import functools

import jax
import jax.numpy as jnp
from jax import lax
from jax.experimental import pallas as pl
from jax.experimental.pallas import tpu as pltpu
from jax.experimental.pallas import tpu_sc as plsc

N_TASKS = 100000
N_DATA = 100000
N_EDGES = 1600000
HID = 32
N_LAYERS = 2
N_REL = 4

IDX_W = 128
CHUNK_K = 4
ROWS_PER_TILE = 784
EP = IDX_W * ROWS_PER_TILE * 16
NROWS = EP // IDX_W
N_ITERS = ROWS_PER_TILE // CHUNK_K
NACC = 104000
TRASH = 100000
ZSLICE = NACC // 4

BLK = 2000
N_BLKS = N_TASKS // BLK

_P_HI = jax.lax.Precision.HIGHEST


def _dot(x, w):
    return jax.lax.dot_general(x, w, (((1,), (0,)), ((), ())),
                               precision=_P_HI,
                               preferred_element_type=jnp.float32)


def _ln_act(x, scale, bias):
    m = jnp.mean(x, axis=-1, keepdims=True)
    v = jnp.mean((x - m) ** 2, axis=-1, keepdims=True)
    return (x - m) / jnp.sqrt(v + 1e-5) * scale + bias


def _silu(x):
    return x * jax.nn.sigmoid(x)


def _stem_body(xt_in, xd_in, wt, bt, wd, bd, lts, ltb, lds, ldb, wpt, bpt,
               wpd, bpd, xt_out, xd_out, t_out):
    xt = _silu(_ln_act(_dot(xt_in[...], wt[...]) + bt[...], lts[...], ltb[...]))
    xd = _silu(_ln_act(_dot(xd_in[...], wd[...]) + bd[...], lds[...], ldb[...]))
    xt_out[...] = xt
    xd_out[...] = xd
    pt = jnp.maximum(_dot(xt, wpt[...]) + bpt[...], 0.0)
    pd = jnp.maximum(_dot(xd, wpd[...]) + bpd[...], 0.0)
    t_out[...] = jnp.concatenate([pt, pd], axis=-1)


def _stem(x_tasks, x_data, wt, bt, wd, bd, lts, ltb, lds, ldb, wpt, bpt,
          wpd, bpd):
    whole = lambda shape: pl.BlockSpec(shape, lambda i: (0,) * len(shape))
    return pl.pallas_call(
        _stem_body,
        grid=(N_BLKS,),
        in_specs=[
            pl.BlockSpec((BLK, 12), lambda i: (i, 0)),
            pl.BlockSpec((BLK, 5), lambda i: (i, 0)),
            whole((12, HID)), whole((1, HID)),
            whole((5, HID)), whole((1, HID)),
            whole((1, HID)), whole((1, HID)),
            whole((1, HID)), whole((1, HID)),
            whole((HID, 96)), whole((1, 96)),
            whole((HID, HID)), whole((1, HID)),
        ],
        out_specs=[
            pl.BlockSpec((BLK, HID), lambda i: (i, 0)),
            pl.BlockSpec((BLK, HID), lambda i: (i, 0)),
            pl.BlockSpec((BLK, 128), lambda i: (i, 0)),
        ],
        out_shape=[
            jax.ShapeDtypeStruct((N_TASKS, HID), jnp.float32),
            jax.ShapeDtypeStruct((N_DATA, HID), jnp.float32),
            jax.ShapeDtypeStruct((N_TASKS, 128), jnp.float32),
        ],
        compiler_params=pltpu.CompilerParams(
            dimension_semantics=("arbitrary",)),
    )(x_tasks, x_data, wt, bt, wd, bd, lts, ltb, lds, ldb, wpt, bpt,
      wpd, bpd)


def _make_combine_body(last):
    def body(s_in, c_in, xt_in, xd_in, wlt, blt, wld, bld,
             lts, ltb, lds, ldb, bet, bed, *rest):
        if last:
            (xt_out, xd_out) = rest
        else:
            (wpt, bpt, wpd, bpd, xt_out, xd_out, t_out) = rest
        mean = s_in[...] / jnp.maximum(c_in[...], 1.0)
        out_t = _dot(mean, wlt[...]) + blt[...]
        out_d = _dot(mean, wld[...]) + bld[...]
        xt = _ln_act(xt_in[...] + bet[0, 0] * _silu(out_t), lts[...], ltb[...])
        xd = _ln_act(xd_in[...] + bed[0, 0] * _silu(out_d), lds[...], ldb[...])
        xt_out[...] = xt
        xd_out[...] = xd
        if not last:
            pt = jnp.maximum(_dot(xt, wpt[...]) + bpt[...], 0.0)
            pd = jnp.maximum(_dot(xd, wpd[...]) + bpd[...], 0.0)
            t_out[...] = jnp.concatenate([pt, pd], axis=-1)
    return body


def _combine(last, s, c, xt, xd, wlt, blt, wld, bld, lts, ltb, lds, ldb,
             bet, bed, wpt=None, bpt=None, wpd=None, bpd=None):
    whole = lambda shape: pl.BlockSpec(shape, lambda i: (0,) * len(shape))
    in_specs = [
        pl.BlockSpec((BLK, 128), lambda i: (i, 0)),
        pl.BlockSpec((BLK, 128), lambda i: (i, 0)),
        pl.BlockSpec((BLK, HID), lambda i: (i, 0)),
        pl.BlockSpec((BLK, HID), lambda i: (i, 0)),
        whole((128, HID)), whole((1, HID)),
        whole((128, HID)), whole((1, HID)),
        whole((1, HID)), whole((1, HID)),
        whole((1, HID)), whole((1, HID)),
        whole((1, 1)), whole((1, 1)),
    ]
    args = [s, c, xt, xd, wlt, blt, wld, bld, lts, ltb, lds, ldb, bet, bed]
    out_specs = [
        pl.BlockSpec((BLK, HID), lambda i: (i, 0)),
        pl.BlockSpec((BLK, HID), lambda i: (i, 0)),
    ]
    out_shape = [
        jax.ShapeDtypeStruct((N_TASKS, HID), jnp.float32),
        jax.ShapeDtypeStruct((N_DATA, HID), jnp.float32),
    ]
    if not last:
        in_specs += [whole((HID, 96)), whole((1, 96)),
                     whole((HID, HID)), whole((1, HID))]
        args += [wpt, bpt, wpd, bpd]
        out_specs.append(pl.BlockSpec((BLK, 128), lambda i: (i, 0)))
        out_shape.append(
            jax.ShapeDtypeStruct((N_TASKS, 128), jnp.float32))
    return pl.pallas_call(
        _make_combine_body(last),
        grid=(N_BLKS,),
        in_specs=in_specs,
        out_specs=out_specs,
        out_shape=out_shape,
        compiler_params=pltpu.CompilerParams(
            dimension_semantics=("arbitrary",)),
    )(*args)


def _make_sc_kernel(with_counts):
    mesh = plsc.VectorSubcoreMesh(core_axis_name="c", subcore_axis_name="s",
                                  num_cores=2, num_subcores=16)

    out_type = [jax.ShapeDtypeStruct((NACC, 8, 16), jnp.float32)]
    if with_counts:
        out_type.append(jax.ShapeDtypeStruct((NACC, 8, 16), jnp.float32))

    scratch = dict(
        src_v=pltpu.VMEM((4, CHUNK_K, IDX_W), jnp.int32),
        dst_v=pltpu.VMEM((4, CHUNK_K, IDX_W), jnp.int32),
        rows_v=pltpu.VMEM((2, CHUNK_K, IDX_W, 16), jnp.float32),
        ones_v=pltpu.VMEM((IDX_W, 16), jnp.float32),
        acc=pltpu.VMEM_SHARED((NACC, 16), jnp.float32),
        isem=pltpu.SemaphoreType.DMA,
        gsem=pltpu.SemaphoreType.DMA,
        ssem=pltpu.SemaphoreType.DMA,
    )

    @functools.partial(pl.kernel, mesh=mesh, out_type=out_type,
                       scratch_types=scratch,
                       compiler_params=pltpu.CompilerParams(
                           use_tc_tiling_on_sc=False))
    def sc_kernel(t_hbm, src_hbm, dst_hbm, zr_hbm, ones_hbm, *outs,
                  src_v, dst_v, rows_v, ones_v, acc, isem, gsem, ssem):
        if with_counts:
            (s_hbm, c_hbm) = outs
        else:
            (s_hbm,) = outs
        c = lax.axis_index("c")
        s = lax.axis_index("s")
        base = s * ROWS_PER_TILE
        n_half = N_ITERS // 2

        def zero_acc():
            @pl.when(s < 4)
            def _():
                pltpu.sync_copy(zr_hbm, acc.at[pl.ds(s * ZSLICE, ZSLICE)])

        for r in range(N_REL):
            def idx_copies(g, p):
                row0 = base + g * CHUNK_K
                return (
                    pltpu.make_async_copy(
                        src_hbm.at[r, c, pl.ds(row0, CHUNK_K)],
                        src_v.at[p], isem),
                    pltpu.make_async_copy(
                        dst_hbm.at[r, pl.ds(row0, CHUNK_K)],
                        dst_v.at[p], isem),
                )

            def issue_idx(g, p):
                for d in idx_copies(g, p):
                    d.start()

            def wait_idx(g, p):
                for d in idx_copies(g, p):
                    d.wait()

            def fire_gathers(p, u):
                for j in range(CHUNK_K):
                    pltpu.async_copy(t_hbm.at[src_v.at[u, j]],
                                     rows_v.at[p, j], gsem)

            def drain_gathers(p, u):
                for j in range(CHUNK_K):
                    pltpu.make_async_copy(t_hbm.at[src_v.at[u, j]],
                                          rows_v.at[p, j], gsem).wait()

            def fire_scatters(p, u):
                for j in range(CHUNK_K):
                    pltpu.async_copy(rows_v.at[p, j], acc.at[dst_v.at[u, j]],
                                     ssem, add=True)

            def drain_scatters(p, u):
                for j in range(CHUNK_K):
                    pltpu.make_async_copy(rows_v.at[p, j],
                                          acc.at[dst_v.at[u, j]],
                                          ssem).wait()

            zero_acc()
            plsc.subcore_barrier()
            issue_idx(0, 0)
            issue_idx(1, 1)
            wait_idx(0, 0)
            fire_gathers(0, 0)
            n_quad = N_ITERS // 4

            def body4(h, carry):
                for j4 in range(4):
                    g = 4 * h + j4
                    p, q = j4 % 2, 1 - (j4 % 2)
                    if j4 == 0:
                        @pl.when(h >= 1)
                        def _():
                            drain_scatters(1, 3)
                    else:
                        drain_scatters(q, j4 - 1)
                    if j4 == 3:
                        @pl.when(h <= n_quad - 2)
                        def _():
                            wait_idx(g + 1, 0)
                            fire_gathers(q, 0)
                            issue_idx(g + 2, 1)
                    else:
                        wait_idx(g + 1, j4 + 1)
                        fire_gathers(q, j4 + 1)
                        if j4 == 2:
                            @pl.when(h <= n_quad - 2)
                            def _():
                                issue_idx(g + 2, 0)
                        else:
                            issue_idx(g + 2, (j4 + 2) % 4)
                    drain_gathers(p, j4)
                    fire_scatters(p, j4)
                return carry

            lax.fori_loop(0, n_quad, body4, 0)
            drain_scatters(1, 3)
            plsc.subcore_barrier()
            @pl.when(s < 4)
            def _():
                pltpu.sync_copy(acc.at[pl.ds(s * ZSLICE, ZSLICE)],
                                s_hbm.at[pl.ds(s * ZSLICE, ZSLICE),
                                         2 * r + c])
            plsc.subcore_barrier()

        if with_counts:
            pltpu.sync_copy(ones_hbm, ones_v)
            for k in range(2):
                rel = c * 2 + k

                def cidx_copy(g, p):
                    row0 = base + g * CHUNK_K
                    return pltpu.make_async_copy(
                        dst_hbm.at[rel, pl.ds(row0, CHUNK_K)],
                        dst_v.at[p], isem)

                def fire_cscat(p):
                    for j in range(CHUNK_K):
                        pltpu.async_copy(ones_v, acc.at[dst_v.at[p, j]],
                                         ssem, add=True)

                def drain_cscat(p):
                    for j in range(CHUNK_K):
                        pltpu.make_async_copy(ones_v,
                                              acc.at[dst_v.at[p, j]],
                                              ssem).wait()

                zero_acc()
                plsc.subcore_barrier()
                cidx_copy(0, 0).start()

                def cbody2(h, carry):
                    cidx_copy(2 * h, 0).wait()

                    @pl.when(h >= 1)
                    def _():
                        drain_cscat(1)
                    cidx_copy(2 * h + 1, 1).start()
                    fire_cscat(0)
                    cidx_copy(2 * h + 1, 1).wait()
                    drain_cscat(0)

                    @pl.when(h <= n_half - 2)
                    def _():
                        cidx_copy(2 * h + 2, 0).start()
                    fire_cscat(1)
                    return carry

                lax.fori_loop(0, n_half, cbody2, 0)
                drain_cscat(1)
                plsc.subcore_barrier()

                @pl.when(s < 4)
                def _():
                    pltpu.sync_copy(acc.at[pl.ds(s * ZSLICE, ZSLICE)],
                                    c_hbm.at[pl.ds(s * ZSLICE, ZSLICE),
                                             2 * rel])
                    pltpu.sync_copy(acc.at[pl.ds(s * ZSLICE, ZSLICE)],
                                    c_hbm.at[pl.ds(s * ZSLICE, ZSLICE),
                                             2 * rel + 1])
                plsc.subcore_barrier()

    return sc_kernel


_sc_kernel_cached = functools.lru_cache(maxsize=None)(_make_sc_kernel)


def kernel(x_tasks, x_data, ei_tt_to, ei_tt_from, ei_t_read_d, ei_d_read_t,
           stem_Wt, stem_bt, stem_Wd, stem_bd,
           ln_t_scale, ln_t_bias, ln_d_scale, ln_d_bias,
           Wp, bp, Wl, bl, beta_t, beta_d):
    f32 = jnp.float32
    pad = jnp.stack([jnp.zeros((EP - N_EDGES,), jnp.int32),
                     jnp.full((EP - N_EDGES,), TRASH, jnp.int32)])
    srcs, dsts = [], []
    for r, e in enumerate((ei_tt_to, ei_tt_from, ei_t_read_d, ei_d_read_t)):
        e = jnp.concatenate([e.astype(jnp.int32), pad], axis=1)
        src, dst = e[0], e[1]
        q = src * 8 + 2 * r
        srcs.append(jnp.stack([q, q + 1]).reshape(2, NROWS, IDX_W))
        dsts.append(dst.reshape(NROWS, IDX_W))
    src_idx = jnp.stack(srcs)
    dst_idx = jnp.stack(dsts)

    zr = jnp.zeros((ZSLICE, 16), f32)
    ones = jnp.ones((IDX_W, 16), f32)

    r2 = lambda a: a.reshape(1, HID)
    z32 = jnp.zeros((HID, HID), f32)
    wpt, bpt, wpd, bpd, wlt, blt, wld, bld = [], [], [], [], [], [], [], []
    for l in range(N_LAYERS):
        wpt.append(jnp.concatenate([Wp[l, 0], Wp[l, 1], Wp[l, 2]], axis=1))
        bpt.append(jnp.concatenate([bp[l, 0], bp[l, 1], bp[l, 2]])
                   .reshape(1, 96))
        wpd.append(Wp[l, 3])
        bpd.append(bp[l, 3].reshape(1, HID))
        wlt.append(jnp.concatenate(
            [Wl[l, 0], Wl[l, 1], z32, Wl[l, 3]], axis=0) / 3.0)
        blt.append(((bl[l, 0] + bl[l, 1] + bl[l, 3]) / 3.0).reshape(1, HID))
        wld.append(jnp.concatenate([z32, z32, Wl[l, 2], z32], axis=0))
        bld.append(bl[l, 2].reshape(1, HID))

    xt, xd, t0 = _stem(
        x_tasks, x_data, stem_Wt, r2(stem_bt), stem_Wd, r2(stem_bd),
        r2(ln_t_scale[0]), r2(ln_t_bias[0]),
        r2(ln_d_scale[0]), r2(ln_d_bias[0]),
        wpt[0], bpt[0], wpd[0], bpd[0])

    s0, cdeg = _sc_kernel_cached(True)(
        t0.reshape(8 * N_TASKS, 16), src_idx, dst_idx, zr, ones)
    s0 = s0.reshape(NACC, 128)
    cdeg = cdeg.reshape(NACC, 128)

    xt, xd, t1 = _combine(
        False, s0, cdeg, xt, xd, wlt[0], blt[0], wld[0], bld[0],
        r2(ln_t_scale[1]), r2(ln_t_bias[1]),
        r2(ln_d_scale[1]), r2(ln_d_bias[1]),
        beta_t[0].reshape(1, 1), beta_d[0].reshape(1, 1),
        wpt[1], bpt[1], wpd[1], bpd[1])

    (s1,) = _sc_kernel_cached(False)(
        t1.reshape(8 * N_TASKS, 16), src_idx, dst_idx, zr, ones)
    s1 = s1.reshape(NACC, 128)

    xt, xd = _combine(
        True, s1, cdeg, xt, xd, wlt[1], blt[1], wld[1], bld[1],
        r2(ln_t_scale[2]), r2(ln_t_bias[2]),
        r2(ln_d_scale[2]), r2(ln_d_bias[2]),
        beta_t[1].reshape(1, 1), beta_d[1].reshape(1, 1))

    return jnp.concatenate([xt, xd], axis=0)
